# Initial kernel scaffold; baseline (speedup 1.0000x reference)
import jax
import jax.numpy as jnp
from jax.experimental import pallas as pl


def kernel(traindata, neighbor):
    raise NotImplementedError("write your pallas kernel here")


# speedup vs baseline: 2.6519x; 2.6519x over previous
import functools

import jax
import jax.numpy as jnp
from jax import lax
from jax.experimental import pallas as pl
from jax.experimental.pallas import tpu as pltpu
from jax.experimental.pallas import tpu_sc as plsc

N = 65536
NSEL = 4096
NT = 16
CH = N // NT
TT = 16384
NGRP = NSEL // 4
MINI32 = -2147483648


def _iota():
    return lax.iota(jnp.int32, 16)


def _lane(vec, w):
    return jnp.sum(jnp.where(_iota() == w, vec, 0))


def _sc_body(bt_hbm, nid_hbm, nfl_hbm, td_hbm, out_hbm,
             btv, keys_v, tdv, tmax_v, hist_v, off_v, hall_v, cbuf_v,
             lu_v, li_v, cu_v, ci_v, cu2_v, ci2_v,
             siv, idf, flv, idv, mxv, idx4_v, orow_v,
             sh_hist, sh_cnt, sh_tmax, sh_runs_u, sh_runs_i, sh_si,
             sem):
    w = lax.axis_index("s")
    ione = jnp.ones((16,), jnp.int32)

    ibase = (NT - 1 - w) * CH
    pltpu.sync_copy(bt_hbm.at[pl.ds(ibase, CH)], btv)

    def keyloop(v, c):
        x = btv[pl.ds(v * 16, 16)]
        b = plsc.bitcast(x, jnp.int32)
        ik = jnp.where(b >= 0, b, b ^ 0x7FFFFFFF)
        ik = jnp.where(x == 0.0, 0, ik)
        keys_v[pl.ds(v * 16, 16)] = ik
        return c

    lax.fori_loop(0, CH // 16, keyloop, 0)

    rbase = w * (TT // NT)
    pltpu.sync_copy(td_hbm.at[pl.ds(rbase * 4, (TT // NT) * 4)], tdv)

    def tmaxloop(v, c):
        i0 = _iota() * 4 + v * 64
        c1 = plsc.load_gather(tdv, [i0 + 1])
        c2 = plsc.load_gather(tdv, [i0 + 2])
        c3 = plsc.load_gather(tdv, [i0 + 3])
        tmax_v[pl.ds(v * 16, 16)] = jnp.maximum(jnp.maximum(c1, c2), c3)
        return c

    lax.fori_loop(0, TT // NT // 16, tmaxloop, 0)
    pltpu.sync_copy(tmax_v, sh_tmax.at[pl.ds(rbase, TT // NT)])

    def zero16(ref, nv):
        def zl(v, c):
            ref[pl.ds(v * 16, 16)] = jnp.zeros((16,), jnp.int32)
            return c
        lax.fori_loop(0, nv, zl, 0)

    P = jnp.zeros((16,), jnp.int32)
    rem = jnp.full((16,), NSEL, jnp.int32)
    for r in range(4):
        sh = 24 - 8 * r
        zero16(hist_v, 16)

        def histloop(v, c, _sh=sh, _r=r, _P=P):
            ik = keys_v[pl.ds(v * 16, 16)]
            fb = ik ^ MINI32
            d = lax.shift_right_logical(fb, _sh) & 255
            if _r == 0:
                m = jnp.full((16,), True)
            else:
                m = lax.shift_right_logical(fb, _sh + 8) == \
                    lax.shift_right_logical(_P, _sh + 8)
            occ, lastm = plsc.scan_count(d, mask=m)
            plsc.addupdate_scatter(hist_v, [d], occ, mask=lastm & m)
            return c

        lax.fori_loop(0, CH // 16, histloop, 0)
        pltpu.sync_copy(hist_v.at[pl.ds(0, 256)], sh_hist.at[pl.ds(w * 256, 256)])
        plsc.subcore_barrier()
        pltpu.sync_copy(sh_hist, hall_v)
        plsc.subcore_barrier()

        carry = jnp.zeros((16,), jnp.int32)
        Dv = jnp.zeros((16,), jnp.int32)
        Cb = jnp.zeros((16,), jnp.int32)
        for bv in range(16):
            acc = jnp.zeros((16,), jnp.int32)
            for t in range(NT):
                acc = acc + hall_v[pl.ds(t * 256 + bv * 16, 16)]
            cs = plsc.cumsum(acc) + carry
            lt = cs < rem
            Dv = Dv + jnp.sum(lt.astype(jnp.int32))
            Cb = Cb + jnp.sum(jnp.where(lt, acc, 0))
            carry = carry + jnp.sum(acc)
        P = P | lax.shift_left(Dv, sh)
        rem = rem - Cb

    Ts = P ^ MINI32
    need_eq = jnp.sum(jnp.where(_iota() == 0, rem, 0))

    def cntloop(v, c):
        ik = keys_v[pl.ds(v * 16, 16)]
        return (c[0] + jnp.sum((ik < Ts).astype(jnp.int32)),
                c[1] + jnp.sum((ik == Ts).astype(jnp.int32)))

    nlt, neq = lax.fori_loop(0, CH // 16, cntloop,
                             (jnp.int32(0), jnp.int32(0)))
    cbuf_v[pl.ds(0, 16)] = jnp.zeros((16,), jnp.int32) + nlt
    pltpu.sync_copy(cbuf_v.at[pl.ds(0, 16)], sh_cnt.at[pl.ds(w * 16, 16)])
    cbuf_v[pl.ds(0, 16)] = jnp.zeros((16,), jnp.int32) + neq
    pltpu.sync_copy(cbuf_v.at[pl.ds(0, 16)], sh_cnt.at[pl.ds(256 + w * 16, 16)])
    plsc.subcore_barrier()
    pltpu.sync_copy(sh_cnt, cbuf_v)
    nltV = plsc.load_gather(cbuf_v, [_iota() * 16])
    neqV = plsc.load_gather(cbuf_v, [_iota() * 16 + 256])
    eq_baseV = plsc.cumsum(neqV) - neqV
    sV = nltV + jnp.minimum(jnp.maximum(need_eq - eq_baseV, 0), neqV)
    sel_baseV = plsc.cumsum(sV) - sV
    eq_base = _lane(eq_baseV, w)

    def comploop(v, c):
        eqr, selr = c
        kv = (CH // 16 - 1) - v
        ik = lax.rev(keys_v[pl.ds(kv * 16, 16)], (0,))
        ivec = ibase + kv * 16 + 15 - _iota()
        eqm = (ik == Ts)
        eqi = eqm.astype(jnp.int32)
        eq_rank = eq_base + eqr + (plsc.cumsum(eqi) - eqi)
        sel = (ik < Ts) | (eqm & (eq_rank < need_eq))
        seli = sel.astype(jnp.int32)
        dst = selr + (plsc.cumsum(seli) - seli)
        plsc.store_scatter(lu_v, [dst], ik, mask=sel)
        plsc.store_scatter(li_v, [dst], ivec, mask=sel)
        return (eqr + jnp.sum(eqi), selr + jnp.sum(seli))

    lax.fori_loop(0, CH // 16, comploop, (jnp.int32(0), jnp.int32(0)))
    pltpu.sync_copy(lu_v, sh_runs_u.at[pl.ds(w * CH, CH)])
    pltpu.sync_copy(li_v, sh_runs_i.at[pl.ds(w * CH, CH)])
    plsc.subcore_barrier()

    @pl.when(w == 0)
    def _stage_d():
        base = jnp.int32(0)
        for t in range(NT):
            pltpu.sync_copy(sh_runs_u.at[pl.ds(t * CH, CH)], lu_v)
            pltpu.sync_copy(sh_runs_i.at[pl.ds(t * CH, CH)], li_v)
            s_t = jnp.sum(jnp.where(_iota() == t, sV, 0))

            def cpl(j, c, _base=base, _s=s_t):
                su = lu_v[pl.ds(j * 16, 16)]
                si = li_v[pl.ds(j * 16, 16)]
                loc = j * 16 + _iota()
                mk = loc < _s
                plsc.store_scatter(cu_v, [_base + loc], su, mask=mk)
                plsc.store_scatter(ci_v, [_base + loc], si, mask=mk)
                return c

            lax.fori_loop(0, (s_t + 15) // 16, cpl, 0)
            base = base + s_t

        for p in range(3):
            shp = 11 * p
            src_u, src_i = (cu_v, ci_v) if p % 2 == 0 else (cu2_v, ci2_v)
            dst_u, dst_i = (cu2_v, ci2_v) if p % 2 == 0 else (cu_v, ci_v)
            zero16(hist_v, 128)

            def hl(v, c, _s=shp, _su=src_u):
                ik = _su[pl.ds(v * 16, 16)]
                d = lax.shift_right_logical(ik ^ MINI32, _s) & 2047
                occ, lm = plsc.scan_count(d)
                plsc.addupdate_scatter(hist_v, [d], occ, mask=lm)
                return c

            lax.fori_loop(0, NSEL // 16, hl, 0)

            def pfx(v, c):
                hv = hist_v[pl.ds(v * 16, 16)]
                cs = plsc.cumsum(hv)
                off_v[pl.ds(v * 16, 16)] = c + cs - hv
                return c + jnp.sum(hv)

            lax.fori_loop(0, 128, pfx, jnp.int32(0))

            def pm(v, c, _s=shp, _su=src_u, _si=src_i, _du=dst_u, _di=dst_i):
                uvec = _su[pl.ds(v * 16, 16)]
                ivec = _si[pl.ds(v * 16, 16)]
                d = lax.shift_right_logical(uvec ^ MINI32, _s) & 2047
                occ, lm = plsc.scan_count(d)
                dstv = plsc.load_gather(off_v, [d]) + occ - 1
                plsc.store_scatter(_du, [dstv], uvec)
                plsc.store_scatter(_di, [dstv], ivec)
                plsc.addupdate_scatter(off_v, [d], occ, mask=lm)
                return c

            lax.fori_loop(0, NSEL // 16, pm, 0)

        pltpu.sync_copy(ci2_v, sh_si)

    plsc.subcore_barrier()

    EPT = NSEL // NT
    tbase = (NSEL - EPT) - EPT * w
    pltpu.sync_copy(sh_si.at[pl.ds(tbase, EPT)], siv)
    for ck in range(EPT // 128):
        s0 = ck * 128
        pltpu.async_copy(nid_hbm.at[siv.at[pl.ds(s0, 128)]],
                         idf.at[pl.ds(s0, 128)], sem).wait()
        pltpu.async_copy(nfl_hbm.at[siv.at[pl.ds(s0, 128)]],
                         flv.at[pl.ds(s0, 128)], sem).wait()

    def idloop(v, c):
        idv[pl.ds(v * 16, 16)] = idf[pl.ds(v * 16, 16)].astype(jnp.int32)
        return c

    lax.fori_loop(0, EPT // 16, idloop, 0)
    for ck in range(EPT // 128):
        s0 = ck * 128
        pltpu.async_copy(sh_tmax.at[idv.at[pl.ds(s0, 128)]],
                         mxv.at[pl.ds(s0, 128)], sem).wait()

    for gv in range(EPT // 4 // 16):
        gl = gv * 16 + _iota()
        mxmin = jnp.full((16,), -100000.0, jnp.float32)
        mind = jnp.full((16,), -100, jnp.int32)
        for j in range(4):
            kidx = (EPT - 1) - 4 * gl - j
            idg = plsc.load_gather(idv, [kidx])
            flg = plsc.load_gather(flv, [kidx])
            mxg = plsc.load_gather(mxv, [kidx])
            upd = (flg != 0.0) == (mxg > mxmin)
            mxmin = jnp.where(upd, mxg, mxmin)
            mind = jnp.where(upd, idg, mind)
        mind = jnp.maximum(mind, 0)
        for c4 in range(4):
            plsc.store_scatter(idx4_v, [gl * 4 + c4], mind * 4 + c4)
    for ck in range(EPT // 128):
        s0 = ck * 128
        pltpu.async_copy(td_hbm.at[idx4_v.at[pl.ds(s0, 128)]],
                         orow_v.at[pl.ds(s0, 128)], sem).wait()
    pltpu.sync_copy(orow_v, out_hbm.at[pl.ds(EPT * w, EPT)])


@functools.partial(jax.jit, static_argnums=())
def _run_sc(bt, nid, nfl, td):
    mesh = plsc.VectorSubcoreMesh(core_axis_name="c", subcore_axis_name="s",
                                  num_cores=1)
    f = pl.kernel(
        _sc_body,
        out_type=jax.ShapeDtypeStruct((NSEL,), jnp.float32),
        mesh=mesh,
        compiler_params=pltpu.CompilerParams(needs_layout_passes=False),
        scratch_types=[
            pltpu.VMEM((CH,), jnp.float32),
            pltpu.VMEM((CH,), jnp.int32),
            pltpu.VMEM(((TT // NT) * 4,), jnp.float32),
            pltpu.VMEM((TT // NT,), jnp.float32),
            pltpu.VMEM((2048,), jnp.int32),
            pltpu.VMEM((2048,), jnp.int32),
            pltpu.VMEM((NT * 256,), jnp.int32),
            pltpu.VMEM((2 * NT * 16,), jnp.int32),
            pltpu.VMEM((CH,), jnp.int32),
            pltpu.VMEM((CH,), jnp.int32),
            pltpu.VMEM((NSEL,), jnp.int32),
            pltpu.VMEM((NSEL,), jnp.int32),
            pltpu.VMEM((NSEL,), jnp.int32),
            pltpu.VMEM((NSEL,), jnp.int32),
            pltpu.VMEM((NSEL // NT,), jnp.int32),
            pltpu.VMEM((NSEL // NT,), jnp.float32),
            pltpu.VMEM((NSEL // NT,), jnp.float32),
            pltpu.VMEM((NSEL // NT,), jnp.int32),
            pltpu.VMEM((NSEL // NT,), jnp.float32),
            pltpu.VMEM((NSEL // NT,), jnp.int32),
            pltpu.VMEM((NSEL // NT,), jnp.float32),
            pltpu.VMEM_SHARED((NT * 256,), jnp.int32),
            pltpu.VMEM_SHARED((2 * NT * 16,), jnp.int32),
            pltpu.VMEM_SHARED((TT,), jnp.float32),
            pltpu.VMEM_SHARED((NT * CH,), jnp.int32),
            pltpu.VMEM_SHARED((NT * CH,), jnp.int32),
            pltpu.VMEM_SHARED((NSEL,), jnp.int32),
            pltpu.SemaphoreType.DMA,
        ],
    )
    return f(bt, nid, nfl, td)


def kernel(traindata, neighbor):
    neighbor = jnp.squeeze(neighbor)
    bt = neighbor[:, 2]
    nid = neighbor[:, 0]
    nfl = neighbor[:, 4]
    td = traindata.reshape(-1)
    out = _run_sc(bt, nid, nfl, td)
    return out.reshape(NGRP, 4).astype(jnp.float64)


# speedup vs baseline: 3.1097x; 1.1726x over previous
import functools

import jax
import jax.numpy as jnp
from jax import lax
from jax.experimental import pallas as pl
from jax.experimental.pallas import tpu as pltpu
from jax.experimental.pallas import tpu_sc as plsc

N = 65536
NSEL = 4096
NT = 16
CH = N // NT
TT = 16384
NGRP = NSEL // 4
MINI32 = -2147483648


def _iota():
    return lax.iota(jnp.int32, 16)


def _lane(vec, w):
    return jnp.sum(jnp.where(_iota() == w, vec, 0))


def _sc_body(bt_hbm, nid_hbm, nfl_hbm, td_hbm, out_hbm,
             btv, keys_v, tdv, tmax_v, hist_v, off_v, hall_v, cbuf_v,
             lu_v, li_v, cu_v, ci_v, cu2_v, ci2_v,
             siv, idf, flv, idv, mxv, idx4_v, orow_v,
             sh_hist, sh_cnt, sh_tmax, sh_runs_u, sh_runs_i, sh_si,
             sem):
    w = lax.axis_index("s")
    ione = jnp.ones((16,), jnp.int32)

    ibase = (NT - 1 - w) * CH
    pltpu.sync_copy(bt_hbm.at[pl.ds(ibase, CH)], btv)

    def keyloop(v, c):
        for u in range(4):
            o = v * 64 + u * 16
            x = btv[pl.ds(o, 16)]
            b = plsc.bitcast(x, jnp.int32)
            ik = jnp.where(b >= 0, b, b ^ 0x7FFFFFFF)
            ik = jnp.where(x == 0.0, 0, ik)
            keys_v[pl.ds(o, 16)] = ik
        return c

    lax.fori_loop(0, CH // 64, keyloop, 0)

    rbase = w * (TT // NT)
    pltpu.sync_copy(td_hbm.at[pl.ds(rbase * 4, (TT // NT) * 4)], tdv)

    def tmaxloop(v, c):
        for u in range(2):
            i0 = _iota() * 4 + v * 128 + u * 64
            c1 = plsc.load_gather(tdv, [i0 + 1])
            c2 = plsc.load_gather(tdv, [i0 + 2])
            c3 = plsc.load_gather(tdv, [i0 + 3])
            tmax_v[pl.ds(v * 32 + u * 16, 16)] = jnp.maximum(
                jnp.maximum(c1, c2), c3)
        return c

    lax.fori_loop(0, TT // NT // 32, tmaxloop, 0)
    pltpu.sync_copy(tmax_v, sh_tmax.at[pl.ds(rbase, TT // NT)])

    def zero16(ref, nv):
        def zl(v, c):
            ref[pl.ds(v * 16, 16)] = jnp.zeros((16,), jnp.int32)
            return c
        lax.fori_loop(0, nv, zl, 0)

    P = jnp.zeros((16,), jnp.int32)
    rem = jnp.full((16,), NSEL, jnp.int32)
    for r in range(4):
        sh = 24 - 8 * r
        zero16(hist_v, 16)

        def histloop(v, c, _sh=sh, _r=r, _P=P):
            for u in range(4):
                ik = keys_v[pl.ds(v * 64 + u * 16, 16)]
                fb = ik ^ MINI32
                d = lax.shift_right_logical(fb, _sh) & 255
                if _r == 0:
                    plsc.addupdate_scatter(hist_v, [d], ione)
                else:
                    m = lax.shift_right_logical(fb, _sh + 8) == \
                        lax.shift_right_logical(_P, _sh + 8)
                    plsc.addupdate_scatter(hist_v, [d], ione, mask=m)
            return c

        lax.fori_loop(0, CH // 64, histloop, 0)
        pltpu.sync_copy(hist_v.at[pl.ds(0, 256)], sh_hist.at[pl.ds(w * 256, 256)])
        plsc.subcore_barrier()
        pltpu.sync_copy(sh_hist, hall_v)
        plsc.subcore_barrier()

        carry = jnp.zeros((16,), jnp.int32)
        Dv = jnp.zeros((16,), jnp.int32)
        Cb = jnp.zeros((16,), jnp.int32)
        for bv in range(16):
            acc = jnp.zeros((16,), jnp.int32)
            for t in range(NT):
                acc = acc + hall_v[pl.ds(t * 256 + bv * 16, 16)]
            cs = plsc.cumsum(acc) + carry
            lt = cs < rem
            Dv = Dv + jnp.sum(lt.astype(jnp.int32))
            Cb = Cb + jnp.sum(jnp.where(lt, acc, 0))
            carry = carry + jnp.sum(acc)
        P = P | lax.shift_left(Dv, sh)
        rem = rem - Cb

    Ts = P ^ MINI32
    need_eq = jnp.sum(jnp.where(_iota() == 0, rem, 0))

    def cntloop(v, c):
        a, b = c
        for u in range(4):
            ik = keys_v[pl.ds(v * 64 + u * 16, 16)]
            a = a + jnp.sum((ik < Ts).astype(jnp.int32))
            b = b + jnp.sum((ik == Ts).astype(jnp.int32))
        return (a, b)

    nlt, neq = lax.fori_loop(0, CH // 64, cntloop,
                             (jnp.int32(0), jnp.int32(0)))
    cbuf_v[pl.ds(0, 16)] = jnp.zeros((16,), jnp.int32) + nlt
    pltpu.sync_copy(cbuf_v.at[pl.ds(0, 16)], sh_cnt.at[pl.ds(w * 16, 16)])
    cbuf_v[pl.ds(0, 16)] = jnp.zeros((16,), jnp.int32) + neq
    pltpu.sync_copy(cbuf_v.at[pl.ds(0, 16)], sh_cnt.at[pl.ds(256 + w * 16, 16)])
    plsc.subcore_barrier()
    pltpu.sync_copy(sh_cnt, cbuf_v)
    nltV = plsc.load_gather(cbuf_v, [_iota() * 16])
    neqV = plsc.load_gather(cbuf_v, [_iota() * 16 + 256])
    eq_baseV = plsc.cumsum(neqV) - neqV
    sV = nltV + jnp.minimum(jnp.maximum(need_eq - eq_baseV, 0), neqV)
    eq_base = _lane(eq_baseV, w)

    def comploop(v, c):
        eqr, selr = c
        for u in range(2):
            kv = (CH // 16 - 1) - (v * 2 + u)
            ik = lax.rev(keys_v[pl.ds(kv * 16, 16)], (0,))
            ivec = ibase + kv * 16 + 15 - _iota()
            eqm = (ik == Ts)
            eqi = eqm.astype(jnp.int32)
            eq_rank = eq_base + eqr + (plsc.cumsum(eqi) - eqi)
            sel = (ik < Ts) | (eqm & (eq_rank < need_eq))
            seli = sel.astype(jnp.int32)
            dst = selr + (plsc.cumsum(seli) - seli)
            plsc.store_scatter(lu_v, [dst], ik, mask=sel)
            plsc.store_scatter(li_v, [dst], ivec, mask=sel)
            eqr = eqr + jnp.sum(eqi)
            selr = selr + jnp.sum(seli)
        return (eqr, selr)

    lax.fori_loop(0, CH // 32, comploop, (jnp.int32(0), jnp.int32(0)))
    pltpu.sync_copy(lu_v, sh_runs_u.at[pl.ds(w * CH, CH)])
    pltpu.sync_copy(li_v, sh_runs_i.at[pl.ds(w * CH, CH)])
    plsc.subcore_barrier()

    @pl.when(w == 0)
    def _stage_d():
        base = jnp.int32(0)
        for t in range(NT):
            pltpu.sync_copy(sh_runs_u.at[pl.ds(t * CH, CH)], lu_v)
            pltpu.sync_copy(sh_runs_i.at[pl.ds(t * CH, CH)], li_v)
            s_t = jnp.sum(jnp.where(_iota() == t, sV, 0))

            def cpl(j, c, _base=base, _s=s_t):
                for u in range(2):
                    su = lu_v[pl.ds(j * 32 + u * 16, 16)]
                    si = li_v[pl.ds(j * 32 + u * 16, 16)]
                    loc = j * 32 + u * 16 + _iota()
                    mk = loc < _s
                    plsc.store_scatter(cu_v, [_base + loc], su, mask=mk)
                    plsc.store_scatter(ci_v, [_base + loc], si, mask=mk)
                return c

            lax.fori_loop(0, (s_t + 31) // 32, cpl, 0)
            base = base + s_t

        for p in range(3):
            shp = 11 * p
            src_u, src_i = (cu_v, ci_v) if p % 2 == 0 else (cu2_v, ci2_v)
            dst_u, dst_i = (cu2_v, ci2_v) if p % 2 == 0 else (cu_v, ci_v)
            zero16(hist_v, 128)

            def hl(v, c, _s=shp, _su=src_u):
                for u in range(4):
                    ik = _su[pl.ds(v * 64 + u * 16, 16)]
                    d = lax.shift_right_logical(ik ^ MINI32, _s) & 2047
                    plsc.addupdate_scatter(hist_v, [d], ione)
                return c

            lax.fori_loop(0, NSEL // 64, hl, 0)

            def pfx(v, c):
                for u in range(2):
                    hv = hist_v[pl.ds(v * 32 + u * 16, 16)]
                    cs = plsc.cumsum(hv)
                    off_v[pl.ds(v * 32 + u * 16, 16)] = c + cs - hv
                    c = c + jnp.sum(hv)
                return c

            lax.fori_loop(0, 64, pfx, jnp.int32(0))

            def pm(v, c, _s=shp, _su=src_u, _si=src_i, _du=dst_u, _di=dst_i):
                for u in range(2):
                    uvec = _su[pl.ds(v * 32 + u * 16, 16)]
                    ivec = _si[pl.ds(v * 32 + u * 16, 16)]
                    d = lax.shift_right_logical(uvec ^ MINI32, _s) & 2047
                    occ, lm = plsc.scan_count(d)
                    dstv = plsc.load_gather(off_v, [d]) + occ - 1
                    plsc.store_scatter(_du, [dstv], uvec)
                    plsc.store_scatter(_di, [dstv], ivec)
                    plsc.addupdate_scatter(off_v, [d], occ, mask=lm)
                return c

            lax.fori_loop(0, NSEL // 32, pm, 0)

        pltpu.sync_copy(ci2_v, sh_si)

    plsc.subcore_barrier()

    EPT = NSEL // NT
    tbase = (NSEL - EPT) - EPT * w
    pltpu.sync_copy(sh_si.at[pl.ds(tbase, EPT)], siv)
    for ck in range(EPT // 128):
        s0 = ck * 128
        pltpu.async_copy(nid_hbm.at[siv.at[pl.ds(s0, 128)]],
                         idf.at[pl.ds(s0, 128)], sem).wait()
        pltpu.async_copy(nfl_hbm.at[siv.at[pl.ds(s0, 128)]],
                         flv.at[pl.ds(s0, 128)], sem).wait()

    def idloop(v, c):
        idv[pl.ds(v * 16, 16)] = idf[pl.ds(v * 16, 16)].astype(jnp.int32)
        return c

    lax.fori_loop(0, EPT // 16, idloop, 0)
    for ck in range(EPT // 128):
        s0 = ck * 128
        pltpu.async_copy(sh_tmax.at[idv.at[pl.ds(s0, 128)]],
                         mxv.at[pl.ds(s0, 128)], sem).wait()

    for gv in range(EPT // 4 // 16):
        gl = gv * 16 + _iota()
        mxmin = jnp.full((16,), -100000.0, jnp.float32)
        mind = jnp.full((16,), -100, jnp.int32)
        for j in range(4):
            kidx = (EPT - 1) - 4 * gl - j
            idg = plsc.load_gather(idv, [kidx])
            flg = plsc.load_gather(flv, [kidx])
            mxg = plsc.load_gather(mxv, [kidx])
            upd = (flg != 0.0) == (mxg > mxmin)
            mxmin = jnp.where(upd, mxg, mxmin)
            mind = jnp.where(upd, idg, mind)
        mind = jnp.maximum(mind, 0)
        for c4 in range(4):
            plsc.store_scatter(idx4_v, [gl * 4 + c4], mind * 4 + c4)
    for ck in range(EPT // 128):
        s0 = ck * 128
        pltpu.async_copy(td_hbm.at[idx4_v.at[pl.ds(s0, 128)]],
                         orow_v.at[pl.ds(s0, 128)], sem).wait()
    pltpu.sync_copy(orow_v, out_hbm.at[pl.ds(EPT * w, EPT)])


@functools.partial(jax.jit, static_argnums=())
def _run_sc(bt, nid, nfl, td):
    mesh = plsc.VectorSubcoreMesh(core_axis_name="c", subcore_axis_name="s",
                                  num_cores=1)
    f = pl.kernel(
        _sc_body,
        out_type=jax.ShapeDtypeStruct((NSEL,), jnp.float32),
        mesh=mesh,
        compiler_params=pltpu.CompilerParams(needs_layout_passes=False),
        scratch_types=[
            pltpu.VMEM((CH,), jnp.float32),
            pltpu.VMEM((CH,), jnp.int32),
            pltpu.VMEM(((TT // NT) * 4,), jnp.float32),
            pltpu.VMEM((TT // NT,), jnp.float32),
            pltpu.VMEM((2048,), jnp.int32),
            pltpu.VMEM((2048,), jnp.int32),
            pltpu.VMEM((NT * 256,), jnp.int32),
            pltpu.VMEM((2 * NT * 16,), jnp.int32),
            pltpu.VMEM((CH,), jnp.int32),
            pltpu.VMEM((CH,), jnp.int32),
            pltpu.VMEM((NSEL,), jnp.int32),
            pltpu.VMEM((NSEL,), jnp.int32),
            pltpu.VMEM((NSEL,), jnp.int32),
            pltpu.VMEM((NSEL,), jnp.int32),
            pltpu.VMEM((NSEL // NT,), jnp.int32),
            pltpu.VMEM((NSEL // NT,), jnp.float32),
            pltpu.VMEM((NSEL // NT,), jnp.float32),
            pltpu.VMEM((NSEL // NT,), jnp.int32),
            pltpu.VMEM((NSEL // NT,), jnp.float32),
            pltpu.VMEM((NSEL // NT,), jnp.int32),
            pltpu.VMEM((NSEL // NT,), jnp.float32),
            pltpu.VMEM_SHARED((NT * 256,), jnp.int32),
            pltpu.VMEM_SHARED((2 * NT * 16,), jnp.int32),
            pltpu.VMEM_SHARED((TT,), jnp.float32),
            pltpu.VMEM_SHARED((NT * CH,), jnp.int32),
            pltpu.VMEM_SHARED((NT * CH,), jnp.int32),
            pltpu.VMEM_SHARED((NSEL,), jnp.int32),
            pltpu.SemaphoreType.DMA,
        ],
    )
    return f(bt, nid, nfl, td)


def kernel(traindata, neighbor):
    neighbor = jnp.squeeze(neighbor)
    bt = neighbor[:, 2]
    nid = neighbor[:, 0]
    nfl = neighbor[:, 4]
    td = traindata.reshape(-1)
    out = _run_sc(bt, nid, nfl, td)
    return out.reshape(NGRP, 4).astype(jnp.float64)


# speedup vs baseline: 3.6186x; 1.1636x over previous
import functools

import jax
import jax.numpy as jnp
from jax import lax
from jax.experimental import pallas as pl
from jax.experimental.pallas import tpu as pltpu
from jax.experimental.pallas import tpu_sc as plsc

N = 65536
NSEL = 4096
NT = 16
CH = N // NT
TT = 16384
NGRP = NSEL // 4
MINI32 = -2147483648


def _iota():
    return lax.iota(jnp.int32, 16)


def _lane(vec, w):
    return jnp.sum(jnp.where(_iota() == w, vec, 0))


def _sc_body(bt_hbm, nid_hbm, nfl_hbm, f1_hbm, f2_hbm, f3_hbm, out_hbm,
             btv, keys_v, tmax_v, hist_v, off_v, hall_v, cbuf_v,
             lu_v, li_v, cu_v, ci_v, cu2_v, ci2_v,
             f1_v, f2_v, f3_v, g1_v, g2_v, g3_v,
             siv, idf_v, flv, idv, mxv, mind_v, orow_v,
             sh_hist, sh_cnt, sh_tmax, sh_runs_u, sh_runs_i, sh_si,
             sem):
    w = lax.axis_index("s")
    ione = jnp.ones((16,), jnp.int32)

    ibase = (NT - 1 - w) * CH
    pltpu.sync_copy(bt_hbm.at[pl.ds(ibase, CH)], btv)

    def keyloop(v, c):
        for u in range(4):
            o = v * 64 + u * 16
            x = btv[pl.ds(o, 16)]
            b = plsc.bitcast(x, jnp.int32)
            ik = jnp.where(b >= 0, b, b ^ 0x7FFFFFFF)
            ik = jnp.where(x == 0.0, 0, ik)
            keys_v[pl.ds(o, 16)] = ik
        return c

    lax.fori_loop(0, CH // 64, keyloop, 0)

    rbase = w * (TT // NT)
    pltpu.sync_copy(f1_hbm.at[pl.ds(rbase, TT // NT)], f1_v)
    pltpu.sync_copy(f2_hbm.at[pl.ds(rbase, TT // NT)], f2_v)
    pltpu.sync_copy(f3_hbm.at[pl.ds(rbase, TT // NT)], f3_v)

    def tmaxloop(v, c):
        for u in range(2):
            o = v * 32 + u * 16
            c1 = f1_v[pl.ds(o, 16)]
            c2 = f2_v[pl.ds(o, 16)]
            c3 = f3_v[pl.ds(o, 16)]
            tmax_v[pl.ds(o, 16)] = jnp.maximum(jnp.maximum(c1, c2), c3)
        return c

    lax.fori_loop(0, TT // NT // 32, tmaxloop, 0)
    pltpu.sync_copy(tmax_v, sh_tmax.at[pl.ds(rbase, TT // NT)])

    def zero16(ref, nv):
        def zl(v, c):
            ref[pl.ds(v * 16, 16)] = jnp.zeros((16,), jnp.int32)
            return c
        lax.fori_loop(0, nv, zl, 0)

    P = jnp.zeros((16,), jnp.int32)
    rem = jnp.full((16,), NSEL, jnp.int32)
    for r in range(4):
        sh = 24 - 8 * r
        zero16(hist_v, 16)

        def histloop(v, c, _sh=sh, _r=r, _P=P):
            for u in range(4):
                ik = keys_v[pl.ds(v * 64 + u * 16, 16)]
                fb = ik ^ MINI32
                d = lax.shift_right_logical(fb, _sh) & 255
                if _r == 0:
                    plsc.addupdate_scatter(hist_v, [d], ione)
                else:
                    m = lax.shift_right_logical(fb, _sh + 8) == \
                        lax.shift_right_logical(_P, _sh + 8)
                    plsc.addupdate_scatter(hist_v, [d], ione, mask=m)
            return c

        lax.fori_loop(0, CH // 64, histloop, 0)
        pltpu.sync_copy(hist_v.at[pl.ds(0, 256)], sh_hist.at[pl.ds(w * 256, 256)])
        plsc.subcore_barrier()
        pltpu.sync_copy(sh_hist, hall_v)
        plsc.subcore_barrier()

        carry = jnp.zeros((16,), jnp.int32)
        Dv = jnp.zeros((16,), jnp.int32)
        Cb = jnp.zeros((16,), jnp.int32)
        for bv in range(16):
            acc = jnp.zeros((16,), jnp.int32)
            for t in range(NT):
                acc = acc + hall_v[pl.ds(t * 256 + bv * 16, 16)]
            cs = plsc.cumsum(acc) + carry
            lt = cs < rem
            Dv = Dv + jnp.sum(lt.astype(jnp.int32))
            Cb = Cb + jnp.sum(jnp.where(lt, acc, 0))
            carry = carry + jnp.sum(acc)
        P = P | lax.shift_left(Dv, sh)
        rem = rem - Cb

    Ts = P ^ MINI32
    need_eq = jnp.sum(jnp.where(_iota() == 0, rem, 0))

    def cntloop(v, c):
        a, b = c
        for u in range(4):
            ik = keys_v[pl.ds(v * 64 + u * 16, 16)]
            a = a + jnp.sum((ik < Ts).astype(jnp.int32))
            b = b + jnp.sum((ik == Ts).astype(jnp.int32))
        return (a, b)

    nlt, neq = lax.fori_loop(0, CH // 64, cntloop,
                             (jnp.int32(0), jnp.int32(0)))
    cbuf_v[pl.ds(0, 16)] = jnp.zeros((16,), jnp.int32) + nlt
    pltpu.sync_copy(cbuf_v.at[pl.ds(0, 16)], sh_cnt.at[pl.ds(w * 16, 16)])
    cbuf_v[pl.ds(0, 16)] = jnp.zeros((16,), jnp.int32) + neq
    pltpu.sync_copy(cbuf_v.at[pl.ds(0, 16)], sh_cnt.at[pl.ds(256 + w * 16, 16)])
    plsc.subcore_barrier()
    pltpu.sync_copy(sh_cnt, cbuf_v)
    nltV = plsc.load_gather(cbuf_v, [_iota() * 16])
    neqV = plsc.load_gather(cbuf_v, [_iota() * 16 + 256])
    eq_baseV = plsc.cumsum(neqV) - neqV
    sV = nltV + jnp.minimum(jnp.maximum(need_eq - eq_baseV, 0), neqV)
    eq_base = _lane(eq_baseV, w)

    def comploop(v, c):
        eqr, selr = c
        for u in range(2):
            kv = (CH // 16 - 1) - (v * 2 + u)
            ik = lax.rev(keys_v[pl.ds(kv * 16, 16)], (0,))
            ivec = ibase + kv * 16 + 15 - _iota()
            eqm = (ik == Ts)
            eqi = eqm.astype(jnp.int32)
            eq_rank = eq_base + eqr + (plsc.cumsum(eqi) - eqi)
            sel = (ik < Ts) | (eqm & (eq_rank < need_eq))
            seli = sel.astype(jnp.int32)
            dst = selr + (plsc.cumsum(seli) - seli)
            plsc.store_scatter(lu_v, [dst], ik, mask=sel)
            plsc.store_scatter(li_v, [dst], ivec, mask=sel)
            eqr = eqr + jnp.sum(eqi)
            selr = selr + jnp.sum(seli)
        return (eqr, selr)

    lax.fori_loop(0, CH // 32, comploop, (jnp.int32(0), jnp.int32(0)))
    pltpu.sync_copy(lu_v, sh_runs_u.at[pl.ds(w * CH, CH)])
    pltpu.sync_copy(li_v, sh_runs_i.at[pl.ds(w * CH, CH)])
    plsc.subcore_barrier()

    @pl.when(w == 0)
    def _stage_d():
        base = jnp.int32(0)
        for t in range(NT):
            pltpu.sync_copy(sh_runs_u.at[pl.ds(t * CH, CH)], lu_v)
            pltpu.sync_copy(sh_runs_i.at[pl.ds(t * CH, CH)], li_v)
            s_t = jnp.sum(jnp.where(_iota() == t, sV, 0))

            def cpl(j, c, _base=base, _s=s_t):
                for u in range(2):
                    su = lu_v[pl.ds(j * 32 + u * 16, 16)]
                    si = li_v[pl.ds(j * 32 + u * 16, 16)]
                    loc = j * 32 + u * 16 + _iota()
                    mk = loc < _s
                    plsc.store_scatter(cu_v, [_base + loc], su, mask=mk)
                    plsc.store_scatter(ci_v, [_base + loc], si, mask=mk)
                return c

            lax.fori_loop(0, (s_t + 31) // 32, cpl, 0)
            base = base + s_t

        for p in range(3):
            shp = 11 * p
            src_u, src_i = (cu_v, ci_v) if p % 2 == 0 else (cu2_v, ci2_v)
            dst_u, dst_i = (cu2_v, ci2_v) if p % 2 == 0 else (cu_v, ci_v)
            zero16(hist_v, 128)

            def hl(v, c, _s=shp, _su=src_u):
                for u in range(4):
                    ik = _su[pl.ds(v * 64 + u * 16, 16)]
                    d = lax.shift_right_logical(ik ^ MINI32, _s) & 2047
                    plsc.addupdate_scatter(hist_v, [d], ione)
                return c

            lax.fori_loop(0, NSEL // 64, hl, 0)

            def pfx(v, c):
                for u in range(2):
                    hv = hist_v[pl.ds(v * 32 + u * 16, 16)]
                    cs = plsc.cumsum(hv)
                    off_v[pl.ds(v * 32 + u * 16, 16)] = c + cs - hv
                    c = c + jnp.sum(hv)
                return c

            lax.fori_loop(0, 64, pfx, jnp.int32(0))

            def pm(v, c, _s=shp, _su=src_u, _si=src_i, _du=dst_u, _di=dst_i):
                for u in range(2):
                    uvec = _su[pl.ds(v * 32 + u * 16, 16)]
                    ivec = _si[pl.ds(v * 32 + u * 16, 16)]
                    d = lax.shift_right_logical(uvec ^ MINI32, _s) & 2047
                    occ, lm = plsc.scan_count(d)
                    dstv = plsc.load_gather(off_v, [d]) + occ - 1
                    plsc.store_scatter(_du, [dstv], uvec)
                    plsc.store_scatter(_di, [dstv], ivec)
                    plsc.addupdate_scatter(off_v, [d], occ, mask=lm)
                return c

            lax.fori_loop(0, NSEL // 32, pm, 0)

        pltpu.sync_copy(ci2_v, sh_si)

    plsc.subcore_barrier()

    EPT = NSEL // NT
    tbase = (NSEL - EPT) - EPT * w
    pltpu.sync_copy(sh_si.at[pl.ds(tbase, EPT)], siv)
    _pend = []
    for ck in range(EPT // 128):
        s0 = ck * 128
        _pend.append(pltpu.async_copy(nid_hbm.at[siv.at[pl.ds(s0, 128)]],
                                      idf_v.at[pl.ds(s0, 128)], sem))
        _pend.append(pltpu.async_copy(nfl_hbm.at[siv.at[pl.ds(s0, 128)]],
                                      flv.at[pl.ds(s0, 128)], sem))
    for h in _pend:
        h.wait()

    def idloop(v, c):
        idv[pl.ds(v * 16, 16)] = idf_v[pl.ds(v * 16, 16)].astype(jnp.int32)
        return c

    lax.fori_loop(0, EPT // 16, idloop, 0)
    for ck in range(EPT // 128):
        s0 = ck * 128
        pltpu.async_copy(sh_tmax.at[idv.at[pl.ds(s0, 128)]],
                         mxv.at[pl.ds(s0, 128)], sem).wait()

    for gv in range(EPT // 4 // 16):
        gl = gv * 16 + _iota()
        mxmin = jnp.full((16,), -100000.0, jnp.float32)
        mind = jnp.full((16,), -100, jnp.int32)
        for j in range(4):
            kidx = (EPT - 1) - 4 * gl - j
            idg = plsc.load_gather(idv, [kidx])
            flg = plsc.load_gather(flv, [kidx])
            mxg = plsc.load_gather(mxv, [kidx])
            upd = (flg != 0.0) == (mxg > mxmin)
            mxmin = jnp.where(upd, mxg, mxmin)
            mind = jnp.where(upd, idg, mind)
        mind = jnp.maximum(mind, 0)
        mind_v[pl.ds(gv * 16, 16)] = mind
    _pend2 = [pltpu.async_copy(f1_hbm.at[mind_v], g1_v, sem),
              pltpu.async_copy(f2_hbm.at[mind_v], g2_v, sem),
              pltpu.async_copy(f3_hbm.at[mind_v], g3_v, sem)]
    for h in _pend2:
        h.wait()
    for gv in range(EPT // 4 // 16):
        gl4 = (gv * 16 + _iota()) * 4
        idw = mind_v[pl.ds(gv * 16, 16)]
        plsc.store_scatter(orow_v, [gl4], idw.astype(jnp.float32))
        plsc.store_scatter(orow_v, [gl4 + 1], g1_v[pl.ds(gv * 16, 16)])
        plsc.store_scatter(orow_v, [gl4 + 2], g2_v[pl.ds(gv * 16, 16)])
        plsc.store_scatter(orow_v, [gl4 + 3], g3_v[pl.ds(gv * 16, 16)])
    pltpu.sync_copy(orow_v, out_hbm.at[pl.ds(EPT * w, EPT)])


@functools.partial(jax.jit, static_argnums=())
def _run_sc(bt, nid, nfl, f1, f2, f3):
    mesh = plsc.VectorSubcoreMesh(core_axis_name="c", subcore_axis_name="s",
                                  num_cores=1)
    f = pl.kernel(
        _sc_body,
        out_type=jax.ShapeDtypeStruct((NSEL,), jnp.float32),
        mesh=mesh,
        compiler_params=pltpu.CompilerParams(needs_layout_passes=False),
        scratch_types=[
            pltpu.VMEM((CH,), jnp.float32),
            pltpu.VMEM((CH,), jnp.int32),
            pltpu.VMEM((TT // NT,), jnp.float32),
            pltpu.VMEM((2048,), jnp.int32),
            pltpu.VMEM((2048,), jnp.int32),
            pltpu.VMEM((NT * 256,), jnp.int32),
            pltpu.VMEM((2 * NT * 16,), jnp.int32),
            pltpu.VMEM((CH,), jnp.int32),
            pltpu.VMEM((CH,), jnp.int32),
            pltpu.VMEM((NSEL,), jnp.int32),
            pltpu.VMEM((NSEL,), jnp.int32),
            pltpu.VMEM((NSEL,), jnp.int32),
            pltpu.VMEM((NSEL,), jnp.int32),
            pltpu.VMEM((TT // NT,), jnp.float32),
            pltpu.VMEM((TT // NT,), jnp.float32),
            pltpu.VMEM((TT // NT,), jnp.float32),
            pltpu.VMEM((NSEL // NT // 4,), jnp.float32),
            pltpu.VMEM((NSEL // NT // 4,), jnp.float32),
            pltpu.VMEM((NSEL // NT // 4,), jnp.float32),
            pltpu.VMEM((NSEL // NT,), jnp.int32),
            pltpu.VMEM((NSEL // NT,), jnp.float32),
            pltpu.VMEM((NSEL // NT,), jnp.float32),
            pltpu.VMEM((NSEL // NT,), jnp.int32),
            pltpu.VMEM((NSEL // NT,), jnp.float32),
            pltpu.VMEM((NSEL // NT // 4,), jnp.int32),
            pltpu.VMEM((NSEL // NT,), jnp.float32),
            pltpu.VMEM_SHARED((NT * 256,), jnp.int32),
            pltpu.VMEM_SHARED((2 * NT * 16,), jnp.int32),
            pltpu.VMEM_SHARED((TT,), jnp.float32),
            pltpu.VMEM_SHARED((NT * CH,), jnp.int32),
            pltpu.VMEM_SHARED((NT * CH,), jnp.int32),
            pltpu.VMEM_SHARED((NSEL,), jnp.int32),
            pltpu.SemaphoreType.DMA,
        ],
    )
    return f(bt, nid, nfl, f1, f2, f3)


def kernel(traindata, neighbor):
    neighbor = jnp.squeeze(neighbor)
    out = _run_sc(neighbor[:, 2], neighbor[:, 0], neighbor[:, 4],
                  traindata[:, 1], traindata[:, 2], traindata[:, 3])
    return out.reshape(NGRP, 4).astype(jnp.float64)
